# baseline (speedup 1.0000x reference)
import functools

import jax
import jax.numpy as jnp
import numpy as np
from jax import lax
from jax.experimental import pallas as pl
from jax.experimental.pallas import tpu as pltpu
from jax.experimental.pallas import tpu_sc as plsc

N = 50000
E = 800000
H = 64
MB = 1000
EPS = 1e-5

NC = 2
NS = 16
NW = NC * NS
R = 1568
NPAD = NW * R
CAP = E + 8192
CHUNK = 6400
BATCH = 96
PADB = 128

_mesh = plsc.VectorSubcoreMesh(core_axis_name="c", subcore_axis_name="s",
                               num_cores=NC, num_subcores=NS)


def _wid():
    return lax.axis_index("s") * NC + lax.axis_index("c")


NCH = E // CHUNK
assert NCH % 2 == 1 and NCH * CHUNK == E


@functools.partial(
    pl.kernel,
    out_type=(
        jax.ShapeDtypeStruct((NW * CAP,), jnp.int32),
        jax.ShapeDtypeStruct((NW * CAP,), jnp.int32),
        jax.ShapeDtypeStruct((NW * CAP,), jnp.float32),
        jax.ShapeDtypeStruct((NW * 16,), jnp.int32),
    ),
    mesh=_mesh,
    compiler_params=pltpu.CompilerParams(needs_layout_passes=False),
    scratch_types=[
        pltpu.VMEM((2 * CHUNK,), jnp.int32),
        pltpu.VMEM((2 * CHUNK,), jnp.int32),
        pltpu.VMEM((2 * CHUNK,), jnp.float32),
        pltpu.VMEM((2 * (CHUNK + 32),), jnp.int32),
        pltpu.VMEM((2 * (CHUNK + 32),), jnp.int32),
        pltpu.VMEM((2 * (CHUNK + 32),), jnp.float32),
        pltpu.VMEM((16,), jnp.int32),
        pltpu.SemaphoreType.DMA,
        pltpu.SemaphoreType.DMA,
        pltpu.SemaphoreType.DMA,
        pltpu.SemaphoreType.DMA,
    ],
)
def _partition(src_hbm, dst_hbm, attr_hbm, src_p, dl_p, at_p, counts,
               csrc, cdst, cattr, ssrc, sdl, sat, cvec,
               ci0, ci1, co0, co1):
    w = _wid()
    lo = w * R
    ci = [ci0, ci1]
    co = [co0, co1]

    def issue_in(c, p):
        base = c * CHUNK
        cs = pl.ds(p * CHUNK, CHUNK)
        pltpu.async_copy(src_hbm.at[pl.ds(base, CHUNK)], csrc.at[cs], ci[p])
        pltpu.async_copy(dst_hbm.at[pl.ds(base, CHUNK)], cdst.at[cs], ci[p])
        pltpu.async_copy(attr_hbm.at[pl.ds(base, CHUNK)], cattr.at[cs], ci[p])

    def wait_in(p):
        cs = pl.ds(p * CHUNK, CHUNK)
        pltpu.make_async_copy(src_hbm.at[pl.ds(0, CHUNK)], csrc.at[cs],
                              ci[p]).wait()
        pltpu.make_async_copy(dst_hbm.at[pl.ds(0, CHUNK)], cdst.at[cs],
                              ci[p]).wait()
        pltpu.make_async_copy(attr_hbm.at[pl.ds(0, CHUNK)], cattr.at[cs],
                              ci[p]).wait()

    def wait_out(p):
        ps = pl.ds(p * (CHUNK + 32), CHUNK + 16)
        pltpu.make_async_copy(ssrc.at[ps], src_p.at[pl.ds(0, CHUNK + 16)],
                              co[p]).wait()
        pltpu.make_async_copy(sdl.at[ps], dl_p.at[pl.ds(0, CHUNK + 16)],
                              co[p]).wait()
        pltpu.make_async_copy(sat.at[ps], at_p.at[pl.ds(0, CHUNK + 16)],
                              co[p]).wait()

    def process(p, glob):
        def grp4(g4, cntv):
            for u in range(4):
                sl = pl.ds(p * CHUNK + (g4 * 4 + u) * 16, 16)
                d = cdst[sl]
                sv = csrc[sl]
                m = (d >= lo) & (d < lo + R)
                mi = m.astype(jnp.int32)
                pos = cntv + plsc.cumsum(mi) - 1
                plsc.store_scatter(ssrc, [pos], sv >> 2, mask=m)
                plsc.store_scatter(sdl, [pos],
                                   (d - lo) | ((sv & 3) << 12), mask=m)
                plsc.store_scatter(sat, [pos], cattr[sl], mask=m)
                cntv = cntv + plsc.all_reduce_population_count(m)
            return cntv

        sbase = p * (CHUNK + 32)
        cntv = lax.fori_loop(0, CHUNK // 64, grp4,
                             jnp.full((16,), sbase, jnp.int32))
        cnt = cntv[0] - sbase
        ssrc[pl.ds(sbase + cnt, 16)] = jnp.zeros((16,), jnp.int32)
        sdl[pl.ds(sbase + cnt, 16)] = jnp.full((16,), R, jnp.int32)
        sat[pl.ds(sbase + cnt, 16)] = jnp.zeros((16,), jnp.float32)
        cnt8 = (cnt + 7) & jnp.int32(-8)
        woff = pl.multiple_of(w * CAP + glob, 8)
        pltpu.async_copy(ssrc.at[pl.ds(sbase, CHUNK + 16)],
                         src_p.at[pl.ds(woff, CHUNK + 16)], co[p])
        pltpu.async_copy(sdl.at[pl.ds(sbase, CHUNK + 16)],
                         dl_p.at[pl.ds(woff, CHUNK + 16)], co[p])
        pltpu.async_copy(sat.at[pl.ds(sbase, CHUNK + 16)],
                         at_p.at[pl.ds(woff, CHUNK + 16)], co[p])
        return glob + cnt8

    issue_in(jnp.int32(0), 0)

    def body2(k, glob):
        c0 = k * 2
        issue_in(c0 + 1, 1)
        wait_in(0)
        pl.when(k > 0)(lambda: wait_out(0))
        glob = process(0, glob)
        issue_in(c0 + 2, 0)
        wait_in(1)
        pl.when(k > 0)(lambda: wait_out(1))
        glob = process(1, glob)
        return glob

    glob = lax.fori_loop(0, (NCH - 1) // 2, body2, jnp.int32(0))
    wait_in(0)
    wait_out(0)
    glob = process(0, glob)
    wait_out(0)
    wait_out(1)

    def dummy_body(g, _):
        ssrc[pl.ds(g * 16, 16)] = jnp.zeros((16,), jnp.int32)
        sdl[pl.ds(g * 16, 16)] = jnp.full((16,), R, jnp.int32)
        sat[pl.ds(g * 16, 16)] = jnp.zeros((16,), jnp.float32)
        return 0
    lax.fori_loop(0, PADB // 16, dummy_body, 0)
    woff = pl.multiple_of(w * CAP + glob, 8)
    pltpu.sync_copy(ssrc.at[pl.ds(0, PADB)], src_p.at[pl.ds(woff, PADB)])
    pltpu.sync_copy(sdl.at[pl.ds(0, PADB)], dl_p.at[pl.ds(woff, PADB)])
    pltpu.sync_copy(sat.at[pl.ds(0, PADB)], at_p.at[pl.ds(woff, PADB)])

    cvec[...] = jnp.full((16,), glob, jnp.int32)
    pltpu.sync_copy(cvec, counts.at[pl.ds(pl.multiple_of(w * 16, 16), 16)])


@functools.partial(
    pl.kernel,
    out_type=jax.ShapeDtypeStruct((NPAD * H,), jnp.float32),
    mesh=_mesh,
    compiler_params=pltpu.CompilerParams(needs_layout_passes=False),
    scratch_types=[
        pltpu.VMEM(((R + 1) * H,), jnp.float32),
        pltpu.VMEM((3, BATCH), jnp.int32),
        pltpu.VMEM((3, BATCH), jnp.int32),
        pltpu.VMEM((3, BATCH), jnp.float32),
        pltpu.VMEM((2, BATCH, 2 * H), jnp.int32),
        pltpu.VMEM((H,), jnp.float32),
        pltpu.VMEM((H,), jnp.float32),
        pltpu.VMEM((16,), jnp.int32),
        pltpu.SemaphoreType.DMA,
        pltpu.SemaphoreType.DMA,
        pltpu.SemaphoreType.DMA,
        pltpu.SemaphoreType.DMA,
        pltpu.SemaphoreType.DMA,
    ],
)
def _sc_layer(hp_hbm, src_p, dl_p, at_p, counts, wl_hbm, bl_hbm, out,
              aggr, srcb, dlb, atb, rows, wlv, blv, cvec,
              ms0, ms1, ms2, gs0, gs1):
    w = _wid()
    ms = [ms0, ms1, ms2]
    gs = [gs0, gs1]
    neg_inf = jnp.float32(-jnp.inf)

    pltpu.sync_copy(wl_hbm, wlv)
    pltpu.sync_copy(bl_hbm, blv)
    pltpu.sync_copy(counts.at[pl.ds(pl.multiple_of(w * 16, 16), 16)], cvec)
    cnt = jnp.max(cvec[...])
    nb = (cnt + (BATCH - 1)) // BATCH
    last = jnp.maximum(nb - 1, 0)

    def init_body(i, _):
        aggr[pl.ds(i * 16, 16)] = jnp.full((16,), neg_inf, jnp.float32)
        return 0
    lax.fori_loop(0, (R + 1) * H // 16, init_body, 0)

    wlq = [wlv[pl.ds(q * 16, 16)] for q in range(H // 16)]
    blq = [blv[pl.ds(q * 16, 16)] for q in range(H // 16)]

    def issue_meta(t, r):
        off = pl.multiple_of(w * CAP + t * BATCH, 8)
        pltpu.async_copy(src_p.at[pl.ds(off, BATCH)], srcb.at[r], ms[r])
        pltpu.async_copy(dl_p.at[pl.ds(off, BATCH)], dlb.at[r], ms[r])
        pltpu.async_copy(at_p.at[pl.ds(off, BATCH)], atb.at[r], ms[r])

    def wait_meta(r):
        pltpu.make_async_copy(src_p.at[pl.ds(0, BATCH)], srcb.at[r], ms[r]).wait()
        pltpu.make_async_copy(dl_p.at[pl.ds(0, BATCH)], dlb.at[r], ms[r]).wait()
        pltpu.make_async_copy(at_p.at[pl.ds(0, BATCH)], atb.at[r], ms[r]).wait()

    def issue_gather(mr, rr):
        pltpu.async_copy(hp_hbm.at[srcb.at[mr]], rows.at[rr], gs[rr])

    def wait_gather(rr):
        pltpu.make_async_copy(hp_hbm.at[pl.ds(0, BATCH)], rows.at[rr],
                              gs[rr]).wait()

    def compute(mr, rr):
        def grp_body(g, _):
            dl16 = dlb[mr, pl.ds(g * 16, 16)]
            a16 = atb[mr, pl.ds(g * 16, 16)]
            for j in range(16):
                e = g * 16 + j
                dlp = dl16[j]
                base = (dlp & 4095) * H
                hoff = (dlp >> 12) * 32
                av = jnp.full((16,), a16[j])
                for q2 in range(2):
                    xw = rows[rr, e, pl.ds(hoff + q2 * 16, 16)]
                    x32 = plsc.bitcast(xw, jnp.bfloat16)
                    ra, rb = plsc.unpack(
                        x32, format=plsc.PackFormat.INTERLEAVED,
                        preferred_element_type=jnp.float32)
                    for h2, rv in ((0, ra), (1, rb)):
                        q = q2 * 2 + h2
                        ef = av * wlq[q] + blq[q]
                        msg = rv * ef
                        sl = pl.ds(base + q * 16, 16)
                        aggr[sl] = jnp.maximum(aggr[sl], msg)
            return 0
        lax.fori_loop(0, BATCH // 16, grp_body, 0)

    issue_meta(jnp.int32(0), 0)
    wait_meta(0)
    issue_gather(0, 0)
    issue_meta(jnp.minimum(jnp.int32(1), last), 1)

    def body6(k0, _):
        base = k0 * 6
        for i in range(6):
            kk = base + i
            issue_meta(jnp.minimum(kk + 2, last), (i + 2) % 3)
            wait_meta((i + 1) % 3)
            wait_gather(i % 2)
            issue_gather((i + 1) % 3, (i + 1) % 2)
            compute(i % 3, i % 2)
        return 0
    nbu = (nb + 5) // 6
    lax.fori_loop(0, nbu, body6, 0)
    wait_meta(1)
    wait_gather(0)

    def sweep_body(i, _):
        sl = pl.ds(i * 16, 16)
        v = aggr[sl]
        aggr[sl] = jnp.where(v == neg_inf, jnp.float32(0.0), v)
        return 0
    lax.fori_loop(0, R * H // 16, sweep_body, 0)

    pltpu.sync_copy(aggr.at[pl.ds(0, R * H)],
                    out.at[pl.ds(pl.multiple_of(w * R * H, 8), R * H)])


def _enc_body(x_ref, w_ref, b_ref, o_ref, o16_ref):
    z = jnp.dot(x_ref[...], w_ref[...],
                preferred_element_type=jnp.float32) + b_ref[...]
    o_ref[...] = z
    o16_ref[...] = z[:, :H].astype(jnp.bfloat16)


def _encoder(x8, We8, be128):
    return pl.pallas_call(
        _enc_body,
        grid=(N // MB,),
        in_specs=[
            pl.BlockSpec((MB, 8), lambda i: (i, 0)),
            pl.BlockSpec((8, 2 * H), lambda i: (0, 0)),
            pl.BlockSpec((1, 2 * H), lambda i: (0, 0)),
        ],
        out_specs=[pl.BlockSpec((MB, 2 * H), lambda i: (i, 0)),
                   pl.BlockSpec((MB, H), lambda i: (i, 0))],
        out_shape=[jax.ShapeDtypeStruct((N, 2 * H), jnp.float32),
                   jax.ShapeDtypeStruct((N, H), jnp.bfloat16)],
    )(x8, We8, be128.reshape(1, 2 * H))


def _upd_body(h_ref, a_ref, wh_ref, wa_ref, b_ref, g_ref, bb_ref,
              o_ref, o16_ref):
    hb = h_ref[...][:, :H]
    z = (jnp.dot(hb, wh_ref[...], preferred_element_type=jnp.float32)
         + jnp.dot(a_ref[...], wa_ref[...], preferred_element_type=jnp.float32)
         + b_ref[...])
    z = jnp.maximum(z, 0.0)
    z = z * (g_ref[...] / jnp.sqrt(1.0 + EPS)) + bb_ref[...]
    z = jnp.maximum(z, 0.0)
    o_ref[...] = z
    o16_ref[...] = z[:, :H].astype(jnp.bfloat16)


def _update(h128, aggr, Wu, bu, gamma, beta):
    pad = ((0, 0), (0, H))
    return pl.pallas_call(
        _upd_body,
        grid=(N // MB,),
        in_specs=[
            pl.BlockSpec((MB, 2 * H), lambda i: (i, 0)),
            pl.BlockSpec((MB, H), lambda i: (i, 0)),
            pl.BlockSpec((H, 2 * H), lambda i: (0, 0)),
            pl.BlockSpec((H, 2 * H), lambda i: (0, 0)),
            pl.BlockSpec((1, 2 * H), lambda i: (0, 0)),
            pl.BlockSpec((1, 2 * H), lambda i: (0, 0)),
            pl.BlockSpec((1, 2 * H), lambda i: (0, 0)),
        ],
        out_specs=[pl.BlockSpec((MB, 2 * H), lambda i: (i, 0)),
                   pl.BlockSpec((MB, H), lambda i: (i, 0))],
        out_shape=[jax.ShapeDtypeStruct((N, 2 * H), jnp.float32),
                   jax.ShapeDtypeStruct((N, H), jnp.bfloat16)],
    )(h128, aggr, jnp.pad(Wu[:H], pad), jnp.pad(Wu[H:][PERM], pad),
      jnp.pad(bu, (0, H)).reshape(1, 2 * H),
      jnp.pad(gamma, (0, H)).reshape(1, 2 * H),
      jnp.pad(beta, (0, H)).reshape(1, 2 * H))


def _post_body(h_ref, w1_ref, b1_ref, w2_ref, b2_ref, o_ref):
    t = jnp.dot(h_ref[...][:, :H], w1_ref[...],
                preferred_element_type=jnp.float32)
    t = jnp.maximum(t + b1_ref[...], 0.0)
    o_ref[...] = jnp.sum(t * w2_ref[...], axis=1, keepdims=True) + b2_ref[...]


def _post(h, W1, b1, W2, b2):
    return pl.pallas_call(
        _post_body,
        grid=(N // MB,),
        in_specs=[
            pl.BlockSpec((MB, 2 * H), lambda i: (i, 0)),
            pl.BlockSpec((H, H), lambda i: (0, 0)),
            pl.BlockSpec((1, H), lambda i: (0, 0)),
            pl.BlockSpec((1, H), lambda i: (0, 0)),
            pl.BlockSpec((1, 1), lambda i: (0, 0)),
        ],
        out_specs=pl.BlockSpec((MB, 1), lambda i: (i, 0)),
        out_shape=jax.ShapeDtypeStruct((N, 1), jnp.float32),
    )(h, W1, b1.reshape(1, H), W2.reshape(1, H), b2.reshape(1, 1))


PERM = np.concatenate([np.arange(0, 32, 2), np.arange(1, 32, 2),
                       np.arange(32, 64, 2), np.arange(33, 64, 2)])


def _pack_table(h16):
    w = jax.lax.bitcast_convert_type(h16.reshape(N, H // 2, 2), jnp.int32)
    return w.reshape(N // 4, 2 * H)


def kernel(x, edge_index, edge_attr, params):
    src = edge_index[0].astype(jnp.int32)
    dst = edge_index[1].astype(jnp.int32)
    ea = edge_attr[:, 0]

    src_p, dl_p, at_p, counts = _partition(src, dst, ea)

    We, be = params['enc']
    x8 = jnp.pad(x, ((0, 0), (0, 2)))
    We8 = jnp.pad(We, ((0, 2), (0, H)))
    be128 = jnp.pad(be, (0, H))
    h, h16 = _encoder(x8, We8, be128)

    for layer in params['layers']:
        Wl, bl = layer['lin_edge']
        aggr_flat = _sc_layer(_pack_table(h16), src_p, dl_p, at_p, counts,
                              Wl[0][PERM], bl[PERM])
        aggr = aggr_flat.reshape(NPAD, H)[:N]
        Wu, bu = layer['lin_update']
        h, h16 = _update(h, aggr, Wu, bu, layer['bn_gamma'],
                         layer['bn_beta'])

    W1, b1 = params['pp1']
    W2, b2 = params['pp2']
    return _post(h, W1, b1, W2, b2)


# speedup vs baseline: 1.0359x; 1.0359x over previous
import functools

import jax
import jax.numpy as jnp
import numpy as np
from jax import lax
from jax.experimental import pallas as pl
from jax.experimental.pallas import tpu as pltpu
from jax.experimental.pallas import tpu_sc as plsc

N = 50000
E = 800000
H = 64
MB = 1000
EPS = 1e-5

NC = 2
NS = 16
NW = NC * NS
R = 1568
NPAD = NW * R
CAP = E + 8192
CHUNK = 6400
BATCH = 96
PADB = 128
NQ = H // 16

_mesh = plsc.VectorSubcoreMesh(core_axis_name="c", subcore_axis_name="s",
                               num_cores=NC, num_subcores=NS)


def _wid():
    return lax.axis_index("s") * NC + lax.axis_index("c")


NCH = E // CHUNK
assert NCH % 2 == 1 and NCH * CHUNK == E


@functools.partial(
    pl.kernel,
    out_type=(
        jax.ShapeDtypeStruct((NW * CAP,), jnp.int32),
        jax.ShapeDtypeStruct((NW * CAP,), jnp.int32),
        jax.ShapeDtypeStruct((NW * CAP,), jnp.float32),
        jax.ShapeDtypeStruct((NW * 16,), jnp.int32),
    ),
    mesh=_mesh,
    compiler_params=pltpu.CompilerParams(needs_layout_passes=False),
    scratch_types=[
        pltpu.VMEM((2 * CHUNK,), jnp.int32),
        pltpu.VMEM((2 * CHUNK,), jnp.int32),
        pltpu.VMEM((2 * CHUNK,), jnp.float32),
        pltpu.VMEM((2 * (CHUNK + 32),), jnp.int32),
        pltpu.VMEM((2 * (CHUNK + 32),), jnp.int32),
        pltpu.VMEM((2 * (CHUNK + 32),), jnp.float32),
        pltpu.VMEM((16,), jnp.int32),
        pltpu.SemaphoreType.DMA,
        pltpu.SemaphoreType.DMA,
        pltpu.SemaphoreType.DMA,
        pltpu.SemaphoreType.DMA,
    ],
)
def _partition(src_hbm, dst_hbm, attr_hbm, src_p, dl_p, at_p, counts,
               csrc, cdst, cattr, ssrc, sdl, sat, cvec,
               ci0, ci1, co0, co1):
    w = _wid()
    lo = w * R
    ci = [ci0, ci1]
    co = [co0, co1]

    def issue_in(c, p):
        base = c * CHUNK
        cs = pl.ds(p * CHUNK, CHUNK)
        pltpu.async_copy(src_hbm.at[pl.ds(base, CHUNK)], csrc.at[cs], ci[p])
        pltpu.async_copy(dst_hbm.at[pl.ds(base, CHUNK)], cdst.at[cs], ci[p])
        pltpu.async_copy(attr_hbm.at[pl.ds(base, CHUNK)], cattr.at[cs], ci[p])

    def wait_in(p):
        cs = pl.ds(p * CHUNK, CHUNK)
        pltpu.make_async_copy(src_hbm.at[pl.ds(0, CHUNK)], csrc.at[cs],
                              ci[p]).wait()
        pltpu.make_async_copy(dst_hbm.at[pl.ds(0, CHUNK)], cdst.at[cs],
                              ci[p]).wait()
        pltpu.make_async_copy(attr_hbm.at[pl.ds(0, CHUNK)], cattr.at[cs],
                              ci[p]).wait()

    def wait_out(p):
        ps = pl.ds(p * (CHUNK + 32), CHUNK + 16)
        pltpu.make_async_copy(ssrc.at[ps], src_p.at[pl.ds(0, CHUNK + 16)],
                              co[p]).wait()
        pltpu.make_async_copy(sdl.at[ps], dl_p.at[pl.ds(0, CHUNK + 16)],
                              co[p]).wait()
        pltpu.make_async_copy(sat.at[ps], at_p.at[pl.ds(0, CHUNK + 16)],
                              co[p]).wait()

    def process(p, glob):
        def grp4(g4, cntv):
            for u in range(4):
                sl = pl.ds(p * CHUNK + (g4 * 4 + u) * 16, 16)
                d = cdst[sl]
                m = (d >= lo) & (d < lo + R)
                mi = m.astype(jnp.int32)
                pos = cntv + plsc.cumsum(mi) - 1
                plsc.store_scatter(ssrc, [pos], csrc[sl], mask=m)
                plsc.store_scatter(sdl, [pos], d - lo, mask=m)
                plsc.store_scatter(sat, [pos], cattr[sl], mask=m)
                cntv = cntv + plsc.all_reduce_population_count(m)
            return cntv

        sbase = p * (CHUNK + 32)
        cntv = lax.fori_loop(0, CHUNK // 64, grp4,
                             jnp.full((16,), sbase, jnp.int32))
        cnt = cntv[0] - sbase
        ssrc[pl.ds(sbase + cnt, 16)] = jnp.zeros((16,), jnp.int32)
        sdl[pl.ds(sbase + cnt, 16)] = jnp.full((16,), R, jnp.int32)
        sat[pl.ds(sbase + cnt, 16)] = jnp.zeros((16,), jnp.float32)
        cnt8 = (cnt + 7) & jnp.int32(-8)
        woff = pl.multiple_of(w * CAP + glob, 8)
        pltpu.async_copy(ssrc.at[pl.ds(sbase, CHUNK + 16)],
                         src_p.at[pl.ds(woff, CHUNK + 16)], co[p])
        pltpu.async_copy(sdl.at[pl.ds(sbase, CHUNK + 16)],
                         dl_p.at[pl.ds(woff, CHUNK + 16)], co[p])
        pltpu.async_copy(sat.at[pl.ds(sbase, CHUNK + 16)],
                         at_p.at[pl.ds(woff, CHUNK + 16)], co[p])
        return glob + cnt8

    issue_in(jnp.int32(0), 0)

    def body2(k, glob):
        c0 = k * 2
        issue_in(c0 + 1, 1)
        wait_in(0)
        pl.when(k > 0)(lambda: wait_out(0))
        glob = process(0, glob)
        issue_in(c0 + 2, 0)
        wait_in(1)
        pl.when(k > 0)(lambda: wait_out(1))
        glob = process(1, glob)
        return glob

    glob = lax.fori_loop(0, (NCH - 1) // 2, body2, jnp.int32(0))
    wait_in(0)
    wait_out(0)
    glob = process(0, glob)
    wait_out(0)
    wait_out(1)

    def dummy_body(g, _):
        ssrc[pl.ds(g * 16, 16)] = jnp.zeros((16,), jnp.int32)
        sdl[pl.ds(g * 16, 16)] = jnp.full((16,), R, jnp.int32)
        sat[pl.ds(g * 16, 16)] = jnp.zeros((16,), jnp.float32)
        return 0
    lax.fori_loop(0, PADB // 16, dummy_body, 0)
    woff = pl.multiple_of(w * CAP + glob, 8)
    pltpu.sync_copy(ssrc.at[pl.ds(0, PADB)], src_p.at[pl.ds(woff, PADB)])
    pltpu.sync_copy(sdl.at[pl.ds(0, PADB)], dl_p.at[pl.ds(woff, PADB)])
    pltpu.sync_copy(sat.at[pl.ds(0, PADB)], at_p.at[pl.ds(woff, PADB)])

    cvec[...] = jnp.full((16,), glob, jnp.int32)
    pltpu.sync_copy(cvec, counts.at[pl.ds(pl.multiple_of(w * 16, 16), 16)])


@functools.partial(
    pl.kernel,
    out_type=jax.ShapeDtypeStruct((NQ * NPAD * 16,), jnp.float32),
    mesh=_mesh,
    compiler_params=pltpu.CompilerParams(needs_layout_passes=False),
    scratch_types=[
        pltpu.VMEM(((R + 1) * 16,), jnp.float32),
        pltpu.VMEM(((R + 1) * 16,), jnp.float32),
        pltpu.VMEM(((R + 1) * 16,), jnp.float32),
        pltpu.VMEM(((R + 1) * 16,), jnp.float32),
        pltpu.VMEM((3, BATCH), jnp.int32),
        pltpu.VMEM((3, BATCH), jnp.int32),
        pltpu.VMEM((3, BATCH), jnp.float32),
        pltpu.VMEM((2, BATCH, 2 * H), jnp.float32),
        pltpu.VMEM((H,), jnp.float32),
        pltpu.VMEM((H,), jnp.float32),
        pltpu.VMEM((16,), jnp.int32),
        pltpu.SemaphoreType.DMA,
        pltpu.SemaphoreType.DMA,
        pltpu.SemaphoreType.DMA,
        pltpu.SemaphoreType.DMA,
        pltpu.SemaphoreType.DMA,
    ],
)
def _sc_layer(h_hbm, src_p, dl_p, at_p, counts, wl_hbm, bl_hbm, out,
              agg0, agg1, agg2, agg3, srcb, dlb, atb, rows, wlv, blv, cvec,
              ms0, ms1, ms2, gs0, gs1):
    w = _wid()
    ms = [ms0, ms1, ms2]
    gs = [gs0, gs1]
    agg = [agg0, agg1, agg2, agg3]
    neg_inf = jnp.float32(-jnp.inf)

    pltpu.sync_copy(wl_hbm, wlv)
    pltpu.sync_copy(bl_hbm, blv)
    pltpu.sync_copy(counts.at[pl.ds(pl.multiple_of(w * 16, 16), 16)], cvec)
    cnt = jnp.max(cvec[...])
    nb = (cnt + (BATCH - 1)) // BATCH
    last = jnp.maximum(nb - 1, 0)

    def init_body(i, _):
        for q in range(NQ):
            agg[q][pl.ds(i * 16, 16)] = jnp.full((16,), neg_inf, jnp.float32)
        return 0
    lax.fori_loop(0, R + 1, init_body, 0)

    wlq = [wlv[pl.ds(q * 16, 16)] for q in range(NQ)]
    blq = [blv[pl.ds(q * 16, 16)] for q in range(NQ)]

    def issue_meta(t, r):
        off = pl.multiple_of(w * CAP + t * BATCH, 8)
        pltpu.async_copy(src_p.at[pl.ds(off, BATCH)], srcb.at[r], ms[r])
        pltpu.async_copy(dl_p.at[pl.ds(off, BATCH)], dlb.at[r], ms[r])
        pltpu.async_copy(at_p.at[pl.ds(off, BATCH)], atb.at[r], ms[r])

    def wait_meta(r):
        pltpu.make_async_copy(src_p.at[pl.ds(0, BATCH)], srcb.at[r],
                              ms[r]).wait()
        pltpu.make_async_copy(dl_p.at[pl.ds(0, BATCH)], dlb.at[r],
                              ms[r]).wait()
        pltpu.make_async_copy(at_p.at[pl.ds(0, BATCH)], atb.at[r],
                              ms[r]).wait()

    def issue_gather(mr, rr):
        pltpu.async_copy(h_hbm.at[srcb.at[mr]], rows.at[rr], gs[rr])

    def wait_gather(rr):
        pltpu.make_async_copy(h_hbm.at[pl.ds(0, BATCH)], rows.at[rr],
                              gs[rr]).wait()

    def compute(mr, rr):
        def grp_body(g, _):
            dl16 = dlb[mr, pl.ds(g * 16, 16)]
            a16 = atb[mr, pl.ds(g * 16, 16)]
            for j in range(16):
                e = g * 16 + j
                base = dl16[j] * 16
                av = jnp.full((16,), a16[j])
                sl = pl.ds(base, 16)
                for q in range(NQ):
                    ef = av * wlq[q] + blq[q]
                    rv = rows[rr, e, pl.ds(q * 16, 16)]
                    msg = rv * ef
                    agg[q][sl] = jnp.maximum(agg[q][sl], msg)
            return 0
        lax.fori_loop(0, BATCH // 16, grp_body, 0)

    issue_meta(jnp.int32(0), 0)
    wait_meta(0)
    issue_gather(0, 0)
    issue_meta(jnp.minimum(jnp.int32(1), last), 1)

    def body6(k0, _):
        base = k0 * 6
        for i in range(6):
            kk = base + i
            issue_meta(jnp.minimum(kk + 2, last), (i + 2) % 3)
            wait_meta((i + 1) % 3)
            wait_gather(i % 2)
            issue_gather((i + 1) % 3, (i + 1) % 2)
            compute(i % 3, i % 2)
        return 0
    nbu = (nb + 5) // 6
    lax.fori_loop(0, nbu, body6, 0)
    wait_meta(1)
    wait_gather(0)

    def sweep_body(i, _):
        for q in range(NQ):
            sl = pl.ds(i * 16, 16)
            v = agg[q][sl]
            agg[q][sl] = jnp.where(v == neg_inf, jnp.float32(0.0), v)
        return 0
    lax.fori_loop(0, R, sweep_body, 0)

    for q in range(NQ):
        pltpu.sync_copy(
            agg[q].at[pl.ds(0, R * 16)],
            out.at[pl.ds(pl.multiple_of((q * NPAD + w * R) * 16, 8), R * 16)])


def _enc_body(x_ref, w_ref, b_ref, o_ref):
    o_ref[...] = jnp.dot(x_ref[...], w_ref[...],
                         preferred_element_type=jnp.float32) + b_ref[...]


def _encoder(x8, We8, be128):
    return pl.pallas_call(
        _enc_body,
        grid=(N // MB,),
        in_specs=[
            pl.BlockSpec((MB, 8), lambda i: (i, 0)),
            pl.BlockSpec((8, 2 * H), lambda i: (0, 0)),
            pl.BlockSpec((1, 2 * H), lambda i: (0, 0)),
        ],
        out_specs=pl.BlockSpec((MB, 2 * H), lambda i: (i, 0)),
        out_shape=jax.ShapeDtypeStruct((N, 2 * H), jnp.float32),
    )(x8, We8, be128.reshape(1, 2 * H))


def _upd_body(h_ref, a_ref, wu_ref, b_ref, g_ref, bb_ref, o_ref):
    cat = jnp.concatenate([h_ref[...][:, :H], a_ref[...]], axis=1)
    z = jnp.dot(cat, wu_ref[...], preferred_element_type=jnp.float32)
    z = jnp.maximum(z + b_ref[...], 0.0)
    z = z / jnp.sqrt(1.0 + EPS) * g_ref[...] + bb_ref[...]
    o_ref[...] = jnp.maximum(z, 0.0)


def _update(h128, aggr, Wu, bu, gamma, beta):
    return pl.pallas_call(
        _upd_body,
        grid=(N // MB,),
        in_specs=[
            pl.BlockSpec((MB, 2 * H), lambda i: (i, 0)),
            pl.BlockSpec((MB, H), lambda i: (i, 0)),
            pl.BlockSpec((2 * H, 2 * H), lambda i: (0, 0)),
            pl.BlockSpec((1, 2 * H), lambda i: (0, 0)),
            pl.BlockSpec((1, 2 * H), lambda i: (0, 0)),
            pl.BlockSpec((1, 2 * H), lambda i: (0, 0)),
        ],
        out_specs=pl.BlockSpec((MB, 2 * H), lambda i: (i, 0)),
        out_shape=jax.ShapeDtypeStruct((N, 2 * H), jnp.float32),
    )(h128, aggr, jnp.pad(Wu, ((0, 0), (0, H))),
      jnp.pad(bu, (0, H)).reshape(1, 2 * H),
      jnp.pad(gamma, (0, H)).reshape(1, 2 * H),
      jnp.pad(beta, (0, H)).reshape(1, 2 * H))


def _post_body(h_ref, w1_ref, b1_ref, w2_ref, b2_ref, o_ref):
    t = jnp.dot(h_ref[...][:, :H], w1_ref[...],
                preferred_element_type=jnp.float32)
    t = jnp.maximum(t + b1_ref[...], 0.0)
    sres = jnp.dot(t, w2_ref[...], preferred_element_type=jnp.float32)
    o_ref[...] = sres[:, :1] + b2_ref[...]


def _post(h, W1, b1, W2, b2):
    return pl.pallas_call(
        _post_body,
        grid=(N // MB,),
        in_specs=[
            pl.BlockSpec((MB, 2 * H), lambda i: (i, 0)),
            pl.BlockSpec((H, H), lambda i: (0, 0)),
            pl.BlockSpec((1, H), lambda i: (0, 0)),
            pl.BlockSpec((H, 2 * H), lambda i: (0, 0)),
            pl.BlockSpec((1, 1), lambda i: (0, 0)),
        ],
        out_specs=pl.BlockSpec((MB, 1), lambda i: (i, 0)),
        out_shape=jax.ShapeDtypeStruct((N, 1), jnp.float32),
    )(h, W1, b1.reshape(1, H), jnp.pad(W2, ((0, 0), (0, 2 * H - 1))),
      b2.reshape(1, 1))


def kernel(x, edge_index, edge_attr, params):
    src = edge_index[0].astype(jnp.int32)
    dst = edge_index[1].astype(jnp.int32)
    ea = edge_attr[:, 0]

    src_p, dl_p, at_p, counts = _partition(src, dst, ea)

    We, be = params['enc']
    x8 = jnp.pad(x, ((0, 0), (0, 2)))
    We8 = jnp.pad(We, ((0, 2), (0, H)))
    be128 = jnp.pad(be, (0, H))
    h = _encoder(x8, We8, be128)

    for layer in params['layers']:
        Wl, bl = layer['lin_edge']
        out = _sc_layer(h, src_p, dl_p, at_p, counts, Wl[0], bl)
        aggr = jnp.transpose(out.reshape(NQ, NPAD, 16),
                             (1, 0, 2)).reshape(NPAD, H)[:N]
        Wu, bu = layer['lin_update']
        h = _update(h, aggr, Wu, bu, layer['bn_gamma'], layer['bn_beta'])

    W1, b1 = params['pp1']
    W2, b2 = params['pp2']
    return _post(h, W1, b1, W2, b2)


# speedup vs baseline: 1.1095x; 1.0710x over previous
import functools

import jax
import jax.numpy as jnp
import numpy as np
from jax import lax
from jax.experimental import pallas as pl
from jax.experimental.pallas import tpu as pltpu
from jax.experimental.pallas import tpu_sc as plsc

N = 50000
E = 800000
H = 64
MB = 1000
EPS = 1e-5

NC = 2
NS = 16
NW = NC * NS
R = 1568
NPAD = NW * R
CAP = E + 8192
CHUNK = 6400
BATCH = 96
PADB = 128
NQ = H // 16

_mesh = plsc.VectorSubcoreMesh(core_axis_name="c", subcore_axis_name="s",
                               num_cores=NC, num_subcores=NS)


def _wid():
    return lax.axis_index("s") * NC + lax.axis_index("c")


NCH = E // CHUNK
assert NCH % 2 == 1 and NCH * CHUNK == E


@functools.partial(
    pl.kernel,
    out_type=(
        jax.ShapeDtypeStruct((NW * CAP,), jnp.int32),
        jax.ShapeDtypeStruct((NW * CAP,), jnp.int32),
        jax.ShapeDtypeStruct((NW * CAP,), jnp.float32),
        jax.ShapeDtypeStruct((NW * 16,), jnp.int32),
    ),
    mesh=_mesh,
    compiler_params=pltpu.CompilerParams(needs_layout_passes=False),
    scratch_types=[
        pltpu.VMEM((2 * CHUNK,), jnp.int32),
        pltpu.VMEM((2 * CHUNK,), jnp.int32),
        pltpu.VMEM((2 * CHUNK,), jnp.float32),
        pltpu.VMEM((2 * (CHUNK + 32),), jnp.int32),
        pltpu.VMEM((2 * (CHUNK + 32),), jnp.int32),
        pltpu.VMEM((2 * (CHUNK + 32),), jnp.float32),
        pltpu.VMEM((16,), jnp.int32),
        pltpu.SemaphoreType.DMA,
        pltpu.SemaphoreType.DMA,
        pltpu.SemaphoreType.DMA,
        pltpu.SemaphoreType.DMA,
    ],
)
def _partition(src_hbm, dst_hbm, attr_hbm, src_p, dl_p, at_p, counts,
               csrc, cdst, cattr, ssrc, sdl, sat, cvec,
               ci0, ci1, co0, co1):
    w = _wid()
    lo = w * R
    ci = [ci0, ci1]
    co = [co0, co1]

    def issue_in(c, p):
        base = c * CHUNK
        cs = pl.ds(p * CHUNK, CHUNK)
        pltpu.async_copy(src_hbm.at[pl.ds(base, CHUNK)], csrc.at[cs], ci[p])
        pltpu.async_copy(dst_hbm.at[pl.ds(base, CHUNK)], cdst.at[cs], ci[p])
        pltpu.async_copy(attr_hbm.at[pl.ds(base, CHUNK)], cattr.at[cs], ci[p])

    def wait_in(p):
        cs = pl.ds(p * CHUNK, CHUNK)
        pltpu.make_async_copy(src_hbm.at[pl.ds(0, CHUNK)], csrc.at[cs],
                              ci[p]).wait()
        pltpu.make_async_copy(dst_hbm.at[pl.ds(0, CHUNK)], cdst.at[cs],
                              ci[p]).wait()
        pltpu.make_async_copy(attr_hbm.at[pl.ds(0, CHUNK)], cattr.at[cs],
                              ci[p]).wait()

    def wait_out(p):
        ps = pl.ds(p * (CHUNK + 32), CHUNK + 16)
        pltpu.make_async_copy(ssrc.at[ps], src_p.at[pl.ds(0, CHUNK + 16)],
                              co[p]).wait()
        pltpu.make_async_copy(sdl.at[ps], dl_p.at[pl.ds(0, CHUNK + 16)],
                              co[p]).wait()
        pltpu.make_async_copy(sat.at[ps], at_p.at[pl.ds(0, CHUNK + 16)],
                              co[p]).wait()

    def process(p, glob):
        def grp4(g4, cntv):
            for u in range(4):
                sl = pl.ds(p * CHUNK + (g4 * 4 + u) * 16, 16)
                d = cdst[sl]
                m = (d >= lo) & (d < lo + R)
                mi = m.astype(jnp.int32)
                pos = cntv + plsc.cumsum(mi) - 1
                plsc.store_scatter(ssrc, [pos], csrc[sl], mask=m)
                plsc.store_scatter(sdl, [pos], d - lo, mask=m)
                plsc.store_scatter(sat, [pos], cattr[sl], mask=m)
                cntv = cntv + plsc.all_reduce_population_count(m)
            return cntv

        sbase = p * (CHUNK + 32)
        cntv = lax.fori_loop(0, CHUNK // 64, grp4,
                             jnp.full((16,), sbase, jnp.int32))
        cnt = cntv[0] - sbase
        ssrc[pl.ds(sbase + cnt, 16)] = jnp.zeros((16,), jnp.int32)
        sdl[pl.ds(sbase + cnt, 16)] = jnp.full((16,), R, jnp.int32)
        sat[pl.ds(sbase + cnt, 16)] = jnp.zeros((16,), jnp.float32)
        cnt8 = (cnt + 7) & jnp.int32(-8)
        woff = pl.multiple_of(w * CAP + glob, 8)
        pltpu.async_copy(ssrc.at[pl.ds(sbase, CHUNK + 16)],
                         src_p.at[pl.ds(woff, CHUNK + 16)], co[p])
        pltpu.async_copy(sdl.at[pl.ds(sbase, CHUNK + 16)],
                         dl_p.at[pl.ds(woff, CHUNK + 16)], co[p])
        pltpu.async_copy(sat.at[pl.ds(sbase, CHUNK + 16)],
                         at_p.at[pl.ds(woff, CHUNK + 16)], co[p])
        return glob + cnt8

    issue_in(jnp.int32(0), 0)

    def body2(k, glob):
        c0 = k * 2
        issue_in(c0 + 1, 1)
        wait_in(0)
        pl.when(k > 0)(lambda: wait_out(0))
        glob = process(0, glob)
        issue_in(c0 + 2, 0)
        wait_in(1)
        pl.when(k > 0)(lambda: wait_out(1))
        glob = process(1, glob)
        return glob

    glob = lax.fori_loop(0, (NCH - 1) // 2, body2, jnp.int32(0))
    wait_in(0)
    wait_out(0)
    glob = process(0, glob)
    wait_out(0)
    wait_out(1)

    def dummy_body(g, _):
        ssrc[pl.ds(g * 16, 16)] = jnp.zeros((16,), jnp.int32)
        sdl[pl.ds(g * 16, 16)] = jnp.full((16,), R, jnp.int32)
        sat[pl.ds(g * 16, 16)] = jnp.zeros((16,), jnp.float32)
        return 0
    lax.fori_loop(0, PADB // 16, dummy_body, 0)
    woff = pl.multiple_of(w * CAP + glob, 8)
    pltpu.sync_copy(ssrc.at[pl.ds(0, PADB)], src_p.at[pl.ds(woff, PADB)])
    pltpu.sync_copy(sdl.at[pl.ds(0, PADB)], dl_p.at[pl.ds(woff, PADB)])
    pltpu.sync_copy(sat.at[pl.ds(0, PADB)], at_p.at[pl.ds(woff, PADB)])

    cvec[...] = jnp.full((16,), glob, jnp.int32)
    pltpu.sync_copy(cvec, counts.at[pl.ds(pl.multiple_of(w * 16, 16), 16)])


@functools.partial(
    pl.kernel,
    out_type=jax.ShapeDtypeStruct((NPAD * H,), jnp.float32),
    mesh=_mesh,
    compiler_params=pltpu.CompilerParams(needs_layout_passes=False),
    scratch_types=[
        pltpu.VMEM(((R + 1) * H,), jnp.float32),
        pltpu.VMEM((3, BATCH), jnp.int32),
        pltpu.VMEM((3, BATCH), jnp.int32),
        pltpu.VMEM((3, BATCH), jnp.float32),
        pltpu.VMEM((2, BATCH, 2 * H), jnp.float32),
        pltpu.VMEM((H,), jnp.float32),
        pltpu.VMEM((H,), jnp.float32),
        pltpu.VMEM((16,), jnp.int32),
        pltpu.SemaphoreType.DMA,
        pltpu.SemaphoreType.DMA,
        pltpu.SemaphoreType.DMA,
        pltpu.SemaphoreType.DMA,
        pltpu.SemaphoreType.DMA,
    ],
)
def _sc_layer(h_hbm, src_p, dl_p, at_p, counts, wl_hbm, bl_hbm, out,
              aggr, srcb, dlb, atb, rows, wlv, blv, cvec,
              ms0, ms1, ms2, gs0, gs1):
    w = _wid()
    ms = [ms0, ms1, ms2]
    gs = [gs0, gs1]
    neg_inf = jnp.float32(-jnp.inf)

    pltpu.sync_copy(wl_hbm, wlv)
    pltpu.sync_copy(bl_hbm, blv)
    pltpu.sync_copy(counts.at[pl.ds(pl.multiple_of(w * 16, 16), 16)], cvec)
    cnt = jnp.max(cvec[...])
    nb = (cnt + (BATCH - 1)) // BATCH
    last = jnp.maximum(nb - 1, 0)

    def init_body(i, _):
        aggr[pl.ds(i * 16, 16)] = jnp.full((16,), neg_inf, jnp.float32)
        return 0
    lax.fori_loop(0, (R + 1) * H // 16, init_body, 0)

    wlq = [wlv[pl.ds(q * 16, 16)] for q in range(NQ)]
    blq = [blv[pl.ds(q * 16, 16)] for q in range(NQ)]

    def issue_meta(t, r):
        off = pl.multiple_of(w * CAP + t * BATCH, 8)
        pltpu.async_copy(src_p.at[pl.ds(off, BATCH)], srcb.at[r], ms[r])
        pltpu.async_copy(dl_p.at[pl.ds(off, BATCH)], dlb.at[r], ms[r])
        pltpu.async_copy(at_p.at[pl.ds(off, BATCH)], atb.at[r], ms[r])

    def wait_meta(r):
        pltpu.make_async_copy(src_p.at[pl.ds(0, BATCH)], srcb.at[r],
                              ms[r]).wait()
        pltpu.make_async_copy(dl_p.at[pl.ds(0, BATCH)], dlb.at[r],
                              ms[r]).wait()
        pltpu.make_async_copy(at_p.at[pl.ds(0, BATCH)], atb.at[r],
                              ms[r]).wait()

    def issue_gather(mr, rr):
        pltpu.async_copy(h_hbm.at[srcb.at[mr]], rows.at[rr], gs[rr])

    def wait_gather(rr):
        pltpu.make_async_copy(h_hbm.at[pl.ds(0, BATCH)], rows.at[rr],
                              gs[rr]).wait()

    def compute(mr, rr):
        def grp_body(g, _):
            dl16 = dlb[mr, pl.ds(g * 16, 16)]
            a16 = atb[mr, pl.ds(g * 16, 16)]
            for j in range(16):
                e = g * 16 + j
                base = dl16[j] * H
                av = jnp.full((16,), a16[j])
                for q in range(NQ):
                    ef = av * wlq[q] + blq[q]
                    rv = rows[rr, e, pl.ds(q * 16, 16)]
                    msg = rv * ef
                    sl = pl.ds(base + q * 16, 16)
                    aggr[sl] = jnp.maximum(aggr[sl], msg)
            return 0
        lax.fori_loop(0, BATCH // 16, grp_body, 0)

    issue_meta(jnp.int32(0), 0)
    wait_meta(0)
    issue_gather(0, 0)
    issue_meta(jnp.minimum(jnp.int32(1), last), 1)

    def body6(k0, _):
        base = k0 * 6
        for i in range(6):
            kk = base + i
            issue_meta(jnp.minimum(kk + 2, last), (i + 2) % 3)
            wait_meta((i + 1) % 3)
            wait_gather(i % 2)
            issue_gather((i + 1) % 3, (i + 1) % 2)
            compute(i % 3, i % 2)
        return 0
    nbu = (nb + 5) // 6
    lax.fori_loop(0, nbu, body6, 0)
    wait_meta(1)
    wait_gather(0)

    def sweep_body(i, _):
        sl = pl.ds(i * 16, 16)
        v = aggr[sl]
        aggr[sl] = jnp.where(v == neg_inf, jnp.float32(0.0), v)
        return 0
    lax.fori_loop(0, R * H // 16, sweep_body, 0)

    pltpu.sync_copy(aggr.at[pl.ds(0, R * H)],
                    out.at[pl.ds(pl.multiple_of(w * R * H, 8), R * H)])


def _enc_body(x_ref, w_ref, b_ref, o_ref):
    o_ref[...] = jnp.dot(x_ref[...], w_ref[...],
                         preferred_element_type=jnp.float32) + b_ref[...]


def _encoder(x8, We8, be128):
    return pl.pallas_call(
        _enc_body,
        grid=(N // MB,),
        in_specs=[
            pl.BlockSpec((MB, 8), lambda i: (i, 0)),
            pl.BlockSpec((8, 2 * H), lambda i: (0, 0)),
            pl.BlockSpec((1, 2 * H), lambda i: (0, 0)),
        ],
        out_specs=pl.BlockSpec((MB, 2 * H), lambda i: (i, 0)),
        out_shape=jax.ShapeDtypeStruct((N, 2 * H), jnp.float32),
    )(x8, We8, be128.reshape(1, 2 * H))


def _upd_body(h_ref, a_ref, wu_ref, b_ref, g_ref, bb_ref, o_ref):
    cat = jnp.concatenate([h_ref[...][:, :H], a_ref[...]], axis=1)
    z = jnp.dot(cat, wu_ref[...], preferred_element_type=jnp.float32)
    z = jnp.maximum(z + b_ref[...], 0.0)
    z = z / jnp.sqrt(1.0 + EPS) * g_ref[...] + bb_ref[...]
    o_ref[...] = jnp.maximum(z, 0.0)


def _update(h128, aggr, Wu, bu, gamma, beta):
    return pl.pallas_call(
        _upd_body,
        grid=(N // MB,),
        in_specs=[
            pl.BlockSpec((MB, 2 * H), lambda i: (i, 0)),
            pl.BlockSpec((MB, H), lambda i: (i, 0)),
            pl.BlockSpec((2 * H, 2 * H), lambda i: (0, 0)),
            pl.BlockSpec((1, 2 * H), lambda i: (0, 0)),
            pl.BlockSpec((1, 2 * H), lambda i: (0, 0)),
            pl.BlockSpec((1, 2 * H), lambda i: (0, 0)),
        ],
        out_specs=pl.BlockSpec((MB, 2 * H), lambda i: (i, 0)),
        out_shape=jax.ShapeDtypeStruct((N, 2 * H), jnp.float32),
    )(h128, aggr, jnp.pad(Wu, ((0, 0), (0, H))),
      jnp.pad(bu, (0, H)).reshape(1, 2 * H),
      jnp.pad(gamma, (0, H)).reshape(1, 2 * H),
      jnp.pad(beta, (0, H)).reshape(1, 2 * H))


def _post_body(h_ref, w1_ref, b1_ref, w2_ref, b2_ref, o_ref):
    t = jnp.dot(h_ref[...][:, :H], w1_ref[...],
                preferred_element_type=jnp.float32)
    t = jnp.maximum(t + b1_ref[...], 0.0)
    sres = jnp.dot(t, w2_ref[...], preferred_element_type=jnp.float32)
    o_ref[...] = sres[:, :1] + b2_ref[...]


def _post(h, W1, b1, W2, b2):
    return pl.pallas_call(
        _post_body,
        grid=(N // MB,),
        in_specs=[
            pl.BlockSpec((MB, 2 * H), lambda i: (i, 0)),
            pl.BlockSpec((H, H), lambda i: (0, 0)),
            pl.BlockSpec((1, H), lambda i: (0, 0)),
            pl.BlockSpec((H, 2 * H), lambda i: (0, 0)),
            pl.BlockSpec((1, 1), lambda i: (0, 0)),
        ],
        out_specs=pl.BlockSpec((MB, 1), lambda i: (i, 0)),
        out_shape=jax.ShapeDtypeStruct((N, 1), jnp.float32),
    )(h, W1, b1.reshape(1, H), jnp.pad(W2, ((0, 0), (0, 2 * H - 1))),
      b2.reshape(1, 1))


def kernel(x, edge_index, edge_attr, params):
    src = edge_index[0].astype(jnp.int32)
    dst = edge_index[1].astype(jnp.int32)
    ea = edge_attr[:, 0]

    src_p, dl_p, at_p, counts = _partition(src, dst, ea)

    We, be = params['enc']
    x8 = jnp.pad(x, ((0, 0), (0, 2)))
    We8 = jnp.pad(We, ((0, 2), (0, H)))
    be128 = jnp.pad(be, (0, H))
    h = _encoder(x8, We8, be128)

    for layer in params['layers']:
        Wl, bl = layer['lin_edge']
        out = _sc_layer(h, src_p, dl_p, at_p, counts, Wl[0], bl)
        aggr = out.reshape(NPAD, H)[:N]
        Wu, bu = layer['lin_update']
        h = _update(h, aggr, Wu, bu, layer['bn_gamma'], layer['bn_beta'])

    W1, b1 = params['pp1']
    W2, b2 = params['pp2']
    return _post(h, W1, b1, W2, b2)
